# baseline (speedup 1.0000x reference)
import functools

import jax
import jax.numpy as jnp
from jax import lax
from jax.experimental import pallas as pl
from jax.experimental.pallas import tpu as pltpu
from jax.experimental.pallas import tpu_sc as plsc

N = 10000
NNEI = 16
DF = 128
H1 = 64
H2 = 32
D3 = DF * 3
NPAIR = N * NNEI

SC_CORES = 2
SC_SUBCORES = 16
NW = SC_CORES * SC_SUBCORES
CH = 128
PERW = 5120
NPAD = NW * PERW


def _mlp_body(x_ref, w1_ref, b1_ref, w2_ref, b2_ref, w3t_ref, b3_ref,
              dei_ref, etot_ref):
    x = x_ref[...]
    w1 = w1_ref[...]
    w2 = w2_ref[...]
    w3t = w3t_ref[...]
    h1 = jax.nn.sigmoid(
        lax.dot(x, w1, preferred_element_type=jnp.float32) + b1_ref[...])
    h2 = jax.nn.sigmoid(
        lax.dot(h1, w2, preferred_element_type=jnp.float32) + b2_ref[...])
    ei = jnp.sum(h2 * w3t, axis=1, keepdims=True) + b3_ref[...]
    g2 = (h2 * (1.0 - h2)) * w3t
    g1 = lax.dot_general(g2, w2, (((1,), (1,)), ((), ())),
                         preferred_element_type=jnp.float32) * (h1 * (1.0 - h1))
    dei_ref[...] = lax.dot_general(g1, w1, (((1,), (1,)), ((), ())),
                                   preferred_element_type=jnp.float32)

    @pl.when(pl.program_id(0) == 0)
    def _():
        etot_ref[...] = jnp.zeros((1, 1), jnp.float32)

    etot_ref[...] += jnp.sum(ei, axis=(0, 1), keepdims=True)


def _mlp(x, w1, b1, w2, b2, w3t, b3):
    blk = 1000
    grid = N // blk
    return pl.pallas_call(
        _mlp_body,
        grid=(grid,),
        in_specs=[
            pl.BlockSpec((blk, DF), lambda i: (i, 0)),
            pl.BlockSpec((DF, H1), lambda i: (0, 0)),
            pl.BlockSpec((1, H1), lambda i: (0, 0)),
            pl.BlockSpec((H1, H2), lambda i: (0, 0)),
            pl.BlockSpec((1, H2), lambda i: (0, 0)),
            pl.BlockSpec((1, H2), lambda i: (0, 0)),
            pl.BlockSpec((1, 1), lambda i: (0, 0)),
        ],
        out_specs=[
            pl.BlockSpec((blk, DF), lambda i: (i, 0)),
            pl.BlockSpec((1, 1), lambda i: (0, 0)),
        ],
        out_shape=[
            jax.ShapeDtypeStruct((N, DF), jnp.float32),
            jax.ShapeDtypeStruct((1, 1), jnp.float32),
        ],
    )(x, w1, b1, w2, b2, w3t, b3)


def _sc_gather_body(table_hbm, idx_hbm, out_hbm, idx_v, rows_v, sem):
    wid = lax.axis_index("s") * SC_CORES + lax.axis_index("c")
    base0 = wid * PERW

    def body(i, carry):
        base = base0 + i * CH
        pltpu.sync_copy(idx_hbm.at[pl.ds(base, CH)], idx_v)
        pltpu.async_copy(table_hbm.at[idx_v], rows_v, sem).wait()
        pltpu.sync_copy(rows_v, out_hbm.at[pl.ds(base, CH)])
        return carry

    lax.fori_loop(0, PERW // CH, body, 0)


def _sc_gather(table3, idx_pad):
    mesh = plsc.VectorSubcoreMesh(core_axis_name="c", subcore_axis_name="s")
    f = functools.partial(
        pl.kernel,
        mesh=mesh,
        out_type=jax.ShapeDtypeStruct((NPAD, D3), jnp.float32),
        scratch_types=[
            pltpu.VMEM((CH,), jnp.int32),
            pltpu.VMEM((CH, D3), jnp.float32),
            pltpu.SemaphoreType.DMA,
        ],
    )(_sc_gather_body)
    return f(table3, idx_pad)


def _contract_body(g_ref, d_ref, out_ref):
    p = g_ref[...] * d_ref[...]
    j = lax.broadcasted_iota(jnp.int32, (NNEI * D3, 3), 0)
    d = lax.broadcasted_iota(jnp.int32, (NNEI * D3, 3), 1)
    sel = ((j % 3) == d).astype(jnp.float32)
    out_ref[...] = lax.dot(p, sel, preferred_element_type=jnp.float32)


def _contract(g2, d2):
    blk = 200
    grid = N // blk
    w = NNEI * D3
    return pl.pallas_call(
        _contract_body,
        grid=(grid,),
        in_specs=[
            pl.BlockSpec((blk, w), lambda i: (i, 0)),
            pl.BlockSpec((blk, w), lambda i: (i, 0)),
        ],
        out_specs=pl.BlockSpec((blk, 3), lambda i: (i, 0)),
        out_shape=jax.ShapeDtypeStruct((N, 3), jnp.float32),
    )(g2, d2)


def kernel(image, dfeat, neighbor, Egroup_weight, divider, W1, b1, W2, b2,
           W3, b3):
    x = image.reshape(N, DF)
    dei, etot = _mlp(x, W1, b1.reshape(1, H1), W2, b2.reshape(1, H2),
                     W3.reshape(1, H2), b3.reshape(1, 1))
    table = jnp.concatenate([jnp.zeros((1, DF), jnp.float32), dei], axis=0)
    table3 = jnp.repeat(table, 3, axis=1)
    idx = neighbor.reshape(-1).astype(jnp.int32)
    idx_pad = jnp.concatenate(
        [idx, jnp.zeros((NPAD - NPAIR,), jnp.int32)], axis=0)
    g = _sc_gather(table3, idx_pad)
    g2 = g[:NPAIR].reshape(N, NNEI * D3)
    d2 = dfeat.reshape(N, NNEI * D3)
    force = _contract(g2, d2)
    return etot.reshape(1), force.reshape(1, N, 3)


# speedup vs baseline: 1.3254x; 1.3254x over previous
import functools

import jax
import jax.numpy as jnp
from jax import lax
from jax.experimental import pallas as pl
from jax.experimental.pallas import tpu as pltpu
from jax.experimental.pallas import tpu_sc as plsc

N = 10000
NNEI = 16
DF = 128
H1 = 64
H2 = 32
D3 = DF * 3
W6 = NNEI * D3
NV = D3 // 16

SC_CORES = 2
SC_SUBCORES = 16
NW = SC_CORES * SC_SUBCORES

ACH = 8
PAIR_CH = ACH * NNEI
AW = 320
NC_BIG = AW // ACH
NC_SMALL = 10


def _mlp_body(x_ref, w1_ref, b1_ref, w2_ref, b2_ref, w3t_ref, b3_ref,
              dei_ref, etot_ref):
    x = x_ref[...]
    w1 = w1_ref[...]
    w2 = w2_ref[...]
    w3t = w3t_ref[...]
    h1 = jax.nn.sigmoid(
        lax.dot(x, w1, preferred_element_type=jnp.float32) + b1_ref[...])
    h2 = jax.nn.sigmoid(
        lax.dot(h1, w2, preferred_element_type=jnp.float32) + b2_ref[...])
    ei = jnp.sum(h2 * w3t, axis=1, keepdims=True) + b3_ref[...]
    g2 = (h2 * (1.0 - h2)) * w3t
    g1 = lax.dot_general(g2, w2, (((1,), (1,)), ((), ())),
                         preferred_element_type=jnp.float32) * (h1 * (1.0 - h1))
    dei_ref[...] = lax.dot_general(g1, w1, (((1,), (1,)), ((), ())),
                                   preferred_element_type=jnp.float32)

    @pl.when(pl.program_id(0) == 0)
    def _():
        etot_ref[...] = jnp.zeros((1, 1), jnp.float32)

    etot_ref[...] += jnp.sum(ei, axis=(0, 1), keepdims=True)


def _mlp(x, w1, b1, w2, b2, w3t, b3):
    blk = 1000
    grid = N // blk
    return pl.pallas_call(
        _mlp_body,
        grid=(grid,),
        in_specs=[
            pl.BlockSpec((blk, DF), lambda i: (i, 0)),
            pl.BlockSpec((DF, H1), lambda i: (0, 0)),
            pl.BlockSpec((1, H1), lambda i: (0, 0)),
            pl.BlockSpec((H1, H2), lambda i: (0, 0)),
            pl.BlockSpec((1, H2), lambda i: (0, 0)),
            pl.BlockSpec((1, H2), lambda i: (0, 0)),
            pl.BlockSpec((1, 1), lambda i: (0, 0)),
        ],
        out_specs=[
            pl.BlockSpec((blk, DF), lambda i: (i, 0)),
            pl.BlockSpec((1, 1), lambda i: (0, 0)),
        ],
        out_shape=[
            jax.ShapeDtypeStruct((N, DF), jnp.float32),
            jax.ShapeDtypeStruct((1, 1), jnp.float32),
        ],
    )(x, w1, b1, w2, b2, w3t, b3)


def _force_body(table_hbm, idx_hbm, dfeat_hbm, out_hbm,
                idx_all, g_v, d_v, fw_v, gsem, dsem):
    wid = lax.axis_index("s") * SC_CORES + lax.axis_index("c")
    last = wid == NW - 1
    start = AW * wid
    nc = jnp.where(last, NC_SMALL, NC_BIG)
    r0 = 2 * NC_BIG * wid

    pltpu.sync_copy(idx_hbm.at[pl.ds(r0, 2 * NC_BIG)], idx_all)

    lane = lax.iota(jnp.int32, 16)
    ones = jnp.full((16,), 1.0, jnp.float32)
    zeros = jnp.full((16,), 0.0, jnp.float32)
    maskc = [jnp.where(lane % 3 == m, ones, zeros) for m in range(3)]

    def chunk_body(c, carry):
        dd = pltpu.async_copy(dfeat_hbm.at[pl.ds(start + ACH * c, ACH)],
                              d_v, dsem)
        dd.wait()
        for half in range(2):
            gd = pltpu.async_copy(table_hbm.at[idx_all.at[2 * c + half]],
                                  g_v, gsem)
            gd.wait()
            for ah in range(ACH // 2):
                a = half * (ACH // 2) + ah

                def kbody(k, acc):
                    row = ah * NNEI + k
                    return tuple(
                        acc[v] + g_v[row, pl.ds(16 * v, 16)]
                        * d_v[a, pl.ds(k * D3 + 16 * v, 16)]
                        for v in range(NV))

                acc = lax.fori_loop(
                    0, NNEI, kbody,
                    tuple(jnp.zeros((16,), jnp.float32) for _ in range(NV)))
                t = [jnp.zeros((16,), jnp.float32) for _ in range(3)]
                for v in range(NV):
                    for d in range(3):
                        t[d] = t[d] + acc[v] * maskc[(d - v) % 3]
                for d in range(3):
                    fw_v[ACH * c + a, pl.ds(16 * d, 16)] = t[d]
        return carry

    lax.fori_loop(0, nc, chunk_body, 0)

    @pl.when(last)
    def _():
        pltpu.sync_copy(fw_v.at[pl.ds(0, ACH * NC_SMALL)],
                        out_hbm.at[pl.ds(start, ACH * NC_SMALL)])

    @pl.when(jnp.logical_not(last))
    def _():
        pltpu.sync_copy(fw_v, out_hbm.at[pl.ds(start, AW)])


def _force(table3, idx2, dfeat2):
    mesh = plsc.VectorSubcoreMesh(core_axis_name="c", subcore_axis_name="s")
    f = functools.partial(
        pl.kernel,
        mesh=mesh,
        out_type=jax.ShapeDtypeStruct((N, 48), jnp.float32),
        scratch_types=[
            pltpu.VMEM((2 * NC_BIG, PAIR_CH // 2), jnp.int32),
            pltpu.VMEM((PAIR_CH // 2, D3), jnp.float32),
            pltpu.VMEM((ACH, W6), jnp.float32),
            pltpu.VMEM((AW, 48), jnp.float32),
            pltpu.SemaphoreType.DMA,
            pltpu.SemaphoreType.DMA,
        ],
    )(_force_body)
    return f(table3, idx2, dfeat2)


def _finish_body(fw_ref, out_ref):
    j = lax.broadcasted_iota(jnp.int32, (48, 3), 0)
    d = lax.broadcasted_iota(jnp.int32, (48, 3), 1)
    sel = ((j // 16) == d).astype(jnp.float32)
    out_ref[...] = lax.dot(fw_ref[...], sel,
                           preferred_element_type=jnp.float32)


def _finish(fw):
    blk = 2000
    return pl.pallas_call(
        _finish_body,
        grid=(N // blk,),
        in_specs=[pl.BlockSpec((blk, 48), lambda i: (i, 0))],
        out_specs=pl.BlockSpec((blk, 3), lambda i: (i, 0)),
        out_shape=jax.ShapeDtypeStruct((N, 3), jnp.float32),
    )(fw)


def kernel(image, dfeat, neighbor, Egroup_weight, divider, W1, b1, W2, b2,
           W3, b3):
    x = image.reshape(N, DF)
    dei, etot = _mlp(x, W1, b1.reshape(1, H1), W2, b2.reshape(1, H2),
                     W3.reshape(1, H2), b3.reshape(1, 1))
    table = jnp.concatenate([jnp.zeros((1, DF), jnp.float32), dei], axis=0)
    table3 = jnp.repeat(table, 3, axis=1)
    idx2 = neighbor.reshape(-1, PAIR_CH // 2).astype(jnp.int32)
    idx2 = jnp.concatenate(
        [idx2, jnp.zeros((60, PAIR_CH // 2), jnp.int32)], axis=0)
    dfeat2 = dfeat.reshape(N, W6)
    fw = _force(table3, idx2, dfeat2)
    force = _finish(fw)
    return etot.reshape(1), force.reshape(1, N, 3)


# speedup vs baseline: 1.5068x; 1.1369x over previous
import functools

import jax
import jax.numpy as jnp
from jax import lax
from jax.experimental import pallas as pl
from jax.experimental.pallas import tpu as pltpu
from jax.experimental.pallas import tpu_sc as plsc

N = 10000
NNEI = 16
DF = 128
H1 = 64
H2 = 32
D3 = DF * 3
W6 = NNEI * D3
NV = D3 // 16

SC_CORES = 2
SC_SUBCORES = 16
NW = SC_CORES * SC_SUBCORES

ACH = 8
PAIR_CH = ACH * NNEI
AW = 320
NC_BIG = AW // ACH
NC_SMALL = 10


def _mlp_body(x_ref, w1_ref, b1_ref, w2_ref, b2_ref, w3t_ref, b3_ref,
              dei_ref, etot_ref):
    x = x_ref[...]
    w1 = w1_ref[...]
    w2 = w2_ref[...]
    w3t = w3t_ref[...]
    h1 = jax.nn.sigmoid(
        lax.dot(x, w1, preferred_element_type=jnp.float32) + b1_ref[...])
    h2 = jax.nn.sigmoid(
        lax.dot(h1, w2, preferred_element_type=jnp.float32) + b2_ref[...])
    ei = jnp.sum(h2 * w3t, axis=1, keepdims=True) + b3_ref[...]
    g2 = (h2 * (1.0 - h2)) * w3t
    g1 = lax.dot_general(g2, w2, (((1,), (1,)), ((), ())),
                         preferred_element_type=jnp.float32) * (h1 * (1.0 - h1))
    dei_ref[...] = lax.dot_general(g1, w1, (((1,), (1,)), ((), ())),
                                   preferred_element_type=jnp.float32)

    @pl.when(pl.program_id(0) == 0)
    def _():
        etot_ref[...] = jnp.zeros((1, 1), jnp.float32)

    etot_ref[...] += jnp.sum(ei, axis=(0, 1), keepdims=True)


def _mlp(x, w1, b1, w2, b2, w3t, b3):
    blk = 1000
    grid = N // blk
    return pl.pallas_call(
        _mlp_body,
        grid=(grid,),
        in_specs=[
            pl.BlockSpec((blk, DF), lambda i: (i, 0)),
            pl.BlockSpec((DF, H1), lambda i: (0, 0)),
            pl.BlockSpec((1, H1), lambda i: (0, 0)),
            pl.BlockSpec((H1, H2), lambda i: (0, 0)),
            pl.BlockSpec((1, H2), lambda i: (0, 0)),
            pl.BlockSpec((1, H2), lambda i: (0, 0)),
            pl.BlockSpec((1, 1), lambda i: (0, 0)),
        ],
        out_specs=[
            pl.BlockSpec((blk, DF), lambda i: (i, 0)),
            pl.BlockSpec((1, 1), lambda i: (0, 0)),
        ],
        out_shape=[
            jax.ShapeDtypeStruct((N, DF), jnp.float32),
            jax.ShapeDtypeStruct((1, 1), jnp.float32),
        ],
    )(x, w1, b1, w2, b2, w3t, b3)


def _force_body(table_hbm, idx_hbm, dfeat_hbm, out_hbm,
                idx_all, g_v, d_v, fw_v, gsem, dsem):
    wid = lax.axis_index("s") * SC_CORES + lax.axis_index("c")
    last = wid == NW - 1
    start = AW * wid
    nc = jnp.where(last, NC_SMALL, NC_BIG)
    r0 = 2 * NC_BIG * wid

    pltpu.sync_copy(idx_hbm.at[pl.ds(r0, 2 * NC_BIG)], idx_all)

    lane = lax.iota(jnp.int32, 16)
    ones = jnp.full((16,), 1.0, jnp.float32)
    zeros = jnp.full((16,), 0.0, jnp.float32)
    maskc = [jnp.where(lane % 3 == m, ones, zeros) for m in range(3)]
    perm = [jnp.full((16,), 5 * m, jnp.int32)
            + lax.shift_right_logical(11 * (lane + m), 5) for m in range(3)]
    dnums = lax.GatherDimensionNumbers(
        offset_dims=(), collapsed_slice_dims=(0,), start_index_map=(0,))

    def expand(src, m):
        return lax.gather(src, perm[m][:, None], dnums, slice_sizes=(1,),
                          mode=lax.GatherScatterMode.PROMISE_IN_BOUNDS)

    def chunk_body(c, carry):
        dd = pltpu.async_copy(dfeat_hbm.at[pl.ds(start + ACH * c, ACH)],
                              d_v, dsem)
        dd.wait()
        for half in range(2):
            gd = pltpu.async_copy(table_hbm.at[idx_all.at[2 * c + half]],
                                  g_v, gsem)
            gd.wait()
            for ah in range(ACH // 2):
                a = half * (ACH // 2) + ah

                def kbody(k, acc):
                    row = ah * NNEI + k
                    src = [g_v[row, pl.ds(16 * q, 16)] for q in range(DF // 16)]
                    return tuple(
                        acc[v] + expand(src[v // 3], v % 3)
                        * d_v[a, pl.ds(k * D3 + 16 * v, 16)]
                        for v in range(NV))

                acc = lax.fori_loop(
                    0, NNEI, kbody,
                    tuple(jnp.zeros((16,), jnp.float32) for _ in range(NV)))
                t = [jnp.zeros((16,), jnp.float32) for _ in range(3)]
                for v in range(NV):
                    for d in range(3):
                        t[d] = t[d] + acc[v] * maskc[(d - v) % 3]
                for d in range(3):
                    fw_v[ACH * c + a, pl.ds(16 * d, 16)] = t[d]
        return carry

    lax.fori_loop(0, nc, chunk_body, 0)

    @pl.when(last)
    def _():
        pltpu.sync_copy(fw_v.at[pl.ds(0, ACH * NC_SMALL)],
                        out_hbm.at[pl.ds(start, ACH * NC_SMALL)])

    @pl.when(jnp.logical_not(last))
    def _():
        pltpu.sync_copy(fw_v, out_hbm.at[pl.ds(start, AW)])


def _force(table, idx2, dfeat2):
    mesh = plsc.VectorSubcoreMesh(core_axis_name="c", subcore_axis_name="s")
    f = functools.partial(
        pl.kernel,
        mesh=mesh,
        out_type=jax.ShapeDtypeStruct((N, 48), jnp.float32),
        scratch_types=[
            pltpu.VMEM((2 * NC_BIG, PAIR_CH // 2), jnp.int32),
            pltpu.VMEM((PAIR_CH // 2, DF), jnp.float32),
            pltpu.VMEM((ACH, W6), jnp.float32),
            pltpu.VMEM((AW, 48), jnp.float32),
            pltpu.SemaphoreType.DMA,
            pltpu.SemaphoreType.DMA,
        ],
    )(_force_body)
    return f(table, idx2, dfeat2)


def _finish_body(fw_ref, out_ref):
    j = lax.broadcasted_iota(jnp.int32, (48, 3), 0)
    d = lax.broadcasted_iota(jnp.int32, (48, 3), 1)
    sel = ((j // 16) == d).astype(jnp.float32)
    out_ref[...] = lax.dot(fw_ref[...], sel,
                           preferred_element_type=jnp.float32)


def _finish(fw):
    blk = 2000
    return pl.pallas_call(
        _finish_body,
        grid=(N // blk,),
        in_specs=[pl.BlockSpec((blk, 48), lambda i: (i, 0))],
        out_specs=pl.BlockSpec((blk, 3), lambda i: (i, 0)),
        out_shape=jax.ShapeDtypeStruct((N, 3), jnp.float32),
    )(fw)


def kernel(image, dfeat, neighbor, Egroup_weight, divider, W1, b1, W2, b2,
           W3, b3):
    x = image.reshape(N, DF)
    dei, etot = _mlp(x, W1, b1.reshape(1, H1), W2, b2.reshape(1, H2),
                     W3.reshape(1, H2), b3.reshape(1, 1))
    table = jnp.concatenate([jnp.zeros((1, DF), jnp.float32), dei], axis=0)
    idx2 = neighbor.reshape(-1, PAIR_CH // 2).astype(jnp.int32)
    idx2 = jnp.concatenate(
        [idx2, jnp.zeros((60, PAIR_CH // 2), jnp.int32)], axis=0)
    dfeat2 = dfeat.reshape(N, W6)
    fw = _force(table, idx2, dfeat2)
    force = _finish(fw)
    return etot.reshape(1), force.reshape(1, N, 3)


# speedup vs baseline: 2.7126x; 1.8003x over previous
import functools

import jax
import jax.numpy as jnp
from jax import lax
from jax.experimental import pallas as pl
from jax.experimental.pallas import tpu as pltpu
from jax.experimental.pallas import tpu_sc as plsc

N = 10000
NNEI = 16
DF = 128
H1 = 64
H2 = 32
D3 = DF * 3
W6 = NNEI * D3
NV = D3 // 16

SC_CORES = 2
SC_SUBCORES = 16
NW = SC_CORES * SC_SUBCORES

ACH = 8
PAIR_CH = ACH * NNEI
AW = 320
NC_BIG = AW // ACH
NC_SMALL = 10


def _mlp_body(x_ref, w1_ref, b1_ref, w2_ref, b2_ref, w3t_ref, b3_ref,
              dei_ref, etot_ref):
    x = x_ref[...]
    w1 = w1_ref[...]
    w2 = w2_ref[...]
    w3t = w3t_ref[...]

    def sigmoid(z):
        return 0.5 * jnp.tanh(0.5 * z) + 0.5

    def bf(z):
        return z.astype(jnp.bfloat16)

    def bdot(a, b, dims):
        return lax.dot_general(bf(a), bf(b), dims,
                               preferred_element_type=jnp.float32)

    nt = (((1,), (0,)), ((), ()))
    ct = (((1,), (1,)), ((), ()))
    h1 = sigmoid(bdot(x, w1, nt) + b1_ref[...])
    h2 = sigmoid(bdot(h1, w2, nt) + b2_ref[...])
    ei = jnp.sum((bf(h2).astype(jnp.float32)
                  * bf(w3t).astype(jnp.float32)), axis=1,
                 keepdims=True) + b3_ref[...]
    g2 = (h2 * (1.0 - h2)) * w3t
    g1 = bdot(g2, w2, ct) * (h1 * (1.0 - h1))
    dei_ref[...] = bdot(g1, w1, ct)

    @pl.when(pl.program_id(0) == 0)
    def _():
        etot_ref[...] = jnp.zeros((1, 1), jnp.float32)

    etot_ref[...] += jnp.sum(ei, axis=(0, 1), keepdims=True)


def _mlp(x, w1, b1, w2, b2, w3t, b3):
    blk = 1000
    grid = N // blk
    return pl.pallas_call(
        _mlp_body,
        grid=(grid,),
        in_specs=[
            pl.BlockSpec((blk, DF), lambda i: (i, 0)),
            pl.BlockSpec((DF, H1), lambda i: (0, 0)),
            pl.BlockSpec((1, H1), lambda i: (0, 0)),
            pl.BlockSpec((H1, H2), lambda i: (0, 0)),
            pl.BlockSpec((1, H2), lambda i: (0, 0)),
            pl.BlockSpec((1, H2), lambda i: (0, 0)),
            pl.BlockSpec((1, 1), lambda i: (0, 0)),
        ],
        out_specs=[
            pl.BlockSpec((blk, DF), lambda i: (i, 0)),
            pl.BlockSpec((1, 1), lambda i: (0, 0)),
        ],
        out_shape=[
            jax.ShapeDtypeStruct((N, DF), jnp.float32),
            jax.ShapeDtypeStruct((1, 1), jnp.float32),
        ],
    )(x, w1, b1, w2, b2, w3t, b3)


def _force_body(table_hbm, idx_hbm, dfeat_hbm, out_hbm,
                idx_all, g_v, d_v, fw_v, gsem, dsem):
    wid = lax.axis_index("s") * SC_CORES + lax.axis_index("c")
    last = wid == NW - 1
    start = AW * wid
    nc = jnp.where(last, NC_SMALL, NC_BIG)
    r0 = 2 * NC_BIG * wid

    pltpu.sync_copy(idx_hbm.at[pl.ds(r0, 2 * NC_BIG)], idx_all)

    def chunk_body(c, carry):
        dd = pltpu.async_copy(dfeat_hbm.at[pl.ds(start + ACH * c, ACH)],
                              d_v, dsem)
        dd.wait()
        for half in range(2):
            gd = pltpu.async_copy(table_hbm.at[idx_all.at[2 * c + half]],
                                  g_v, gsem)
            gd.wait()
            for ah in range(ACH // 2):
                a = half * (ACH // 2) + ah

                def kbody(k, acc):
                    row = ah * NNEI + k
                    src = [g_v[row, pl.ds(16 * q, 16)] for q in range(DF // 16)]
                    new = []
                    for d in range(3):
                        ad = acc[d]
                        for q in range(DF // 16):
                            ad = ad + src[q] * d_v[
                                a, pl.ds(k * DF + d * (NNEI * DF) + 16 * q, 16)]
                        new.append(ad)
                    return tuple(new)

                acc = lax.fori_loop(
                    0, NNEI, kbody,
                    tuple(jnp.zeros((16,), jnp.float32) for _ in range(3)))
                for d in range(3):
                    fw_v[ACH * c + a, pl.ds(16 * d, 16)] = acc[d]
        return carry

    lax.fori_loop(0, nc, chunk_body, 0)

    @pl.when(last)
    def _():
        pltpu.sync_copy(fw_v.at[pl.ds(0, ACH * NC_SMALL)],
                        out_hbm.at[pl.ds(start, ACH * NC_SMALL)])

    @pl.when(jnp.logical_not(last))
    def _():
        pltpu.sync_copy(fw_v, out_hbm.at[pl.ds(start, AW)])


def _force(table, idx2, dfeat2):
    mesh = plsc.VectorSubcoreMesh(core_axis_name="c", subcore_axis_name="s")
    f = functools.partial(
        pl.kernel,
        mesh=mesh,
        out_type=jax.ShapeDtypeStruct((N, 48), jnp.float32),
        scratch_types=[
            pltpu.VMEM((2 * NC_BIG, PAIR_CH // 2), jnp.int32),
            pltpu.VMEM((PAIR_CH // 2, DF), jnp.float32),
            pltpu.VMEM((ACH, W6), jnp.float32),
            pltpu.VMEM((AW, 48), jnp.float32),
            pltpu.SemaphoreType.DMA,
            pltpu.SemaphoreType.DMA,
        ],
    )(_force_body)
    return f(table, idx2, dfeat2)


def _finish_body(fw_ref, out_ref):
    j = lax.broadcasted_iota(jnp.int32, (48, 3), 0)
    d = lax.broadcasted_iota(jnp.int32, (48, 3), 1)
    sel = ((j // 16) == d).astype(jnp.float32)
    out_ref[...] = lax.dot(fw_ref[...], sel,
                           preferred_element_type=jnp.float32)


def _finish(fw):
    blk = 2000
    return pl.pallas_call(
        _finish_body,
        grid=(N // blk,),
        in_specs=[pl.BlockSpec((blk, 48), lambda i: (i, 0))],
        out_specs=pl.BlockSpec((blk, 3), lambda i: (i, 0)),
        out_shape=jax.ShapeDtypeStruct((N, 3), jnp.float32),
    )(fw)


def kernel(image, dfeat, neighbor, Egroup_weight, divider, W1, b1, W2, b2,
           W3, b3):
    x = image.reshape(N, DF)
    dei, etot = _mlp(x, W1, b1.reshape(1, H1), W2, b2.reshape(1, H2),
                     W3.reshape(1, H2), b3.reshape(1, 1))
    table = jnp.concatenate([jnp.zeros((1, DF), jnp.float32), dei], axis=0)
    idx2 = neighbor.reshape(-1, PAIR_CH // 2).astype(jnp.int32)
    idx2 = jnp.concatenate(
        [idx2, jnp.zeros((60, PAIR_CH // 2), jnp.int32)], axis=0)
    dfeat2 = dfeat.transpose(0, 1, 4, 2, 3).reshape(N, W6)
    fw = _force(table, idx2, dfeat2)
    force = _finish(fw)
    return etot.reshape(1), force.reshape(1, N, 3)


# speedup vs baseline: 6.5496x; 2.4145x over previous
import functools

import jax
import jax.numpy as jnp
from jax import lax
from jax.experimental import pallas as pl
from jax.experimental.pallas import tpu as pltpu
from jax.experimental.pallas import tpu_sc as plsc

N = 10000
NNEI = 16
DF = 128
H1 = 64
H2 = 32
D3 = DF * 3
W6 = NNEI * D3
NV = D3 // 16

SC_CORES = 2
SC_SUBCORES = 16
NW = SC_CORES * SC_SUBCORES

ACH = 8
PAIR_CH = ACH * NNEI
AW = 320
NC_BIG = AW // ACH
NC_SMALL = 10


def _mlp_body(x_ref, w1_ref, b1_ref, w2_ref, b2_ref, w3t_ref, b3_ref,
              dei_ref, etot_ref):
    x = x_ref[...]
    w1 = w1_ref[...]
    w2 = w2_ref[...]
    w3t = w3t_ref[...]

    def sigmoid(z):
        return 0.5 * jnp.tanh(0.5 * z) + 0.5

    def bf(z):
        return z.astype(jnp.bfloat16)

    def bdot(a, b, dims):
        return lax.dot_general(bf(a), bf(b), dims,
                               preferred_element_type=jnp.float32)

    nt = (((1,), (0,)), ((), ()))
    ct = (((1,), (1,)), ((), ()))
    h1 = sigmoid(bdot(x, w1, nt) + b1_ref[...])
    h2 = sigmoid(bdot(h1, w2, nt) + b2_ref[...])
    ei = jnp.sum((bf(h2).astype(jnp.float32)
                  * bf(w3t).astype(jnp.float32)), axis=1,
                 keepdims=True) + b3_ref[...]
    g2 = (h2 * (1.0 - h2)) * w3t
    g1 = bdot(g2, w2, ct) * (h1 * (1.0 - h1))
    dei_ref[...] = bdot(g1, w1, ct)

    @pl.when(pl.program_id(0) == 0)
    def _():
        etot_ref[...] = jnp.zeros((1, 1), jnp.float32)

    etot_ref[...] += jnp.sum(ei, axis=(0, 1), keepdims=True)


def _mlp(x, w1, b1, w2, b2, w3t, b3):
    blk = 1000
    grid = N // blk
    return pl.pallas_call(
        _mlp_body,
        grid=(grid,),
        in_specs=[
            pl.BlockSpec((blk, DF), lambda i: (i, 0)),
            pl.BlockSpec((DF, H1), lambda i: (0, 0)),
            pl.BlockSpec((1, H1), lambda i: (0, 0)),
            pl.BlockSpec((H1, H2), lambda i: (0, 0)),
            pl.BlockSpec((1, H2), lambda i: (0, 0)),
            pl.BlockSpec((1, H2), lambda i: (0, 0)),
            pl.BlockSpec((1, 1), lambda i: (0, 0)),
        ],
        out_specs=[
            pl.BlockSpec((blk, DF), lambda i: (i, 0)),
            pl.BlockSpec((1, 1), lambda i: (0, 0)),
        ],
        out_shape=[
            jax.ShapeDtypeStruct((N, DF), jnp.float32),
            jax.ShapeDtypeStruct((1, 1), jnp.float32),
        ],
    )(x, w1, b1, w2, b2, w3t, b3)


def _force_body(table_hbm, idx_hbm, dfeat_hbm, out_hbm,
                idx_all, g_v, d_v, fw_v, gsem0, gsem1, dsem0, dsem1):
    gsem = [gsem0, gsem1]
    dsem = [dsem0, dsem1]
    wid = lax.axis_index("s") * SC_CORES + lax.axis_index("c")
    last = wid == NW - 1
    start = AW * wid
    nc = jnp.where(last, NC_SMALL, NC_BIG)
    r0 = 2 * NC_BIG * wid

    pltpu.sync_copy(idx_hbm.at[pl.ds(r0, 2 * NC_BIG)], idx_all)

    nu = 2 * nc
    UF = 4 * W6

    def issue(u, s):
        pltpu.async_copy(table_hbm.at[idx_all.at[u]], g_v.at[s], gsem[s])
        pltpu.async_copy(dfeat_hbm.at[pl.ds((start + 4 * u) * W6, UF)],
                         d_v.at[s], dsem[s])

    def drain(u, s):
        pltpu.make_async_copy(table_hbm.at[idx_all.at[u]], g_v.at[s],
                              gsem[s]).wait()
        pltpu.make_async_copy(dfeat_hbm.at[pl.ds((start + 4 * u) * W6, UF)],
                              d_v.at[s], dsem[s]).wait()

    issue(0, 0)
    issue(1, 1)

    def unit_body(up, carry):
        for s in range(2):
            u = 2 * up + s
            drain(u, s)
            for a in range(4):
                def kbody(k, acc):
                    row = a * NNEI + k
                    src = [g_v[s, row, pl.ds(16 * q, 16)]
                           for q in range(DF // 16)]
                    new = []
                    for d in range(3):
                        ad = acc[d]
                        for q in range(DF // 16):
                            ad = ad + src[q] * d_v[
                                s, pl.ds(a * W6 + d * (NNEI * DF)
                                         + k * DF + 16 * q, 16)]
                        new.append(ad)
                    return tuple(new)

                acc = lax.fori_loop(
                    0, NNEI, kbody,
                    tuple(jnp.zeros((16,), jnp.float32) for _ in range(3)))
                for d in range(3):
                    fw_v[4 * u + a, pl.ds(16 * d, 16)] = acc[d]

            @pl.when(u + 2 < nu)
            def _():
                issue(u + 2, s)
        return carry

    lax.fori_loop(0, nc, unit_body, 0)

    @pl.when(last)
    def _():
        pltpu.sync_copy(fw_v.at[pl.ds(0, ACH * NC_SMALL)],
                        out_hbm.at[pl.ds(start, ACH * NC_SMALL)])

    @pl.when(jnp.logical_not(last))
    def _():
        pltpu.sync_copy(fw_v, out_hbm.at[pl.ds(start, AW)])


def _force(table, idx2, dfeat2):
    mesh = plsc.VectorSubcoreMesh(core_axis_name="c", subcore_axis_name="s")
    f = functools.partial(
        pl.kernel,
        mesh=mesh,
        out_type=jax.ShapeDtypeStruct((N, 48), jnp.float32),
        scratch_types=[
            pltpu.VMEM((2 * NC_BIG, PAIR_CH // 2), jnp.int32),
            pltpu.VMEM((2, PAIR_CH // 2, DF), jnp.float32),
            pltpu.VMEM((2, 4 * W6), jnp.float32),
            pltpu.VMEM((AW, 48), jnp.float32),
            pltpu.SemaphoreType.DMA,
            pltpu.SemaphoreType.DMA,
            pltpu.SemaphoreType.DMA,
            pltpu.SemaphoreType.DMA,
        ],
    )(_force_body)
    return f(table, idx2, dfeat2)


def _finish_body(fw_ref, out_ref):
    j = lax.broadcasted_iota(jnp.int32, (48, 3), 0)
    d = lax.broadcasted_iota(jnp.int32, (48, 3), 1)
    sel = ((j // 16) == d).astype(jnp.float32)
    out_ref[...] = lax.dot(fw_ref[...], sel,
                           preferred_element_type=jnp.float32)


def _finish(fw):
    blk = 2000
    return pl.pallas_call(
        _finish_body,
        grid=(N // blk,),
        in_specs=[pl.BlockSpec((blk, 48), lambda i: (i, 0))],
        out_specs=pl.BlockSpec((blk, 3), lambda i: (i, 0)),
        out_shape=jax.ShapeDtypeStruct((N, 3), jnp.float32),
    )(fw)


def kernel(image, dfeat, neighbor, Egroup_weight, divider, W1, b1, W2, b2,
           W3, b3):
    x = image.reshape(N, DF)
    dei, etot = _mlp(x, W1, b1.reshape(1, H1), W2, b2.reshape(1, H2),
                     W3.reshape(1, H2), b3.reshape(1, 1))
    table = jnp.concatenate([jnp.zeros((1, DF), jnp.float32), dei], axis=0)
    idx2 = neighbor.reshape(-1, PAIR_CH // 2).astype(jnp.int32)
    idx2 = jnp.concatenate(
        [idx2, jnp.zeros((60, PAIR_CH // 2), jnp.int32)], axis=0)
    dfeat2 = dfeat.transpose(0, 1, 4, 2, 3).reshape(N * W6)
    fw = _force(table, idx2, dfeat2)
    force = _finish(fw)
    return etot.reshape(1), force.reshape(1, N, 3)
